# Initial kernel scaffold; baseline (speedup 1.0000x reference)
import jax
import jax.numpy as jnp
from jax.experimental import pallas as pl


def kernel(x, lut0, lut1, lut2):
    raise NotImplementedError("write your pallas kernel here")


# speedup vs baseline: 24.9446x; 24.9446x over previous
import functools

import jax
import jax.numpy as jnp
from jax import lax
from jax.experimental import pallas as pl
from jax.experimental.pallas import tpu as pltpu
from jax.experimental.pallas import tpu_sc as plsc

_B = 4096
_H = 200
_F = 32
_CW = 3
_PW = 10
_OUT_LEN = _H - _CW + 1
_POOL_OUT = _OUT_LEN // _PW
_T = _POOL_OUT * _PW + _CW - 1
_D = _CW * _F
_NW = 32
_ROWS = _B // _NW
_OUT_W = _F * _POOL_OUT


def _sc_body(x_hbm, tab_hbm, out_hbm, idx_v, emb_v, orow_v, sem):
    wid = lax.axis_index("s") * 2 + lax.axis_index("c")
    base = wid * _ROWS
    lane = lax.iota(jnp.int32, 16)
    sidx0 = lane * _POOL_OUT
    sidx1 = (lane + 16) * _POOL_OUT

    @pl.loop(0, _ROWS)
    def _row(r):
        row = base + r
        pltpu.sync_copy(x_hbm.at[row, pl.ds(0, _T)], idx_v)
        c0 = pltpu.async_copy(
            tab_hbm.at[idx_v.at[pl.ds(0, 96)]], emb_v.at[pl.ds(0, 96)], sem)
        c1 = pltpu.async_copy(
            tab_hbm.at[idx_v.at[pl.ds(96, 96)]], emb_v.at[pl.ds(96, 96)], sem)
        c0.wait()
        c1.wait()

        @pl.loop(0, _POOL_OUT)
        def _grp(g):
            j0 = g * _PW
            acc0 = jnp.zeros((16,), jnp.float32)
            acc1 = jnp.zeros((16,), jnp.float32)
            for s in range(_PW):
                j = j0 + s
                e0 = emb_v[j, pl.ds(0, 16)]
                e1 = emb_v[j, pl.ds(16, 16)]
                f0 = emb_v[j + 1, pl.ds(32, 16)]
                f1 = emb_v[j + 1, pl.ds(48, 16)]
                g0 = emb_v[j + 2, pl.ds(64, 16)]
                g1 = emb_v[j + 2, pl.ds(80, 16)]
                acc0 = acc0 + jnp.maximum(e0 + f0 + g0, 0.0)
                acc1 = acc1 + jnp.maximum(e1 + f1 + g1, 0.0)
            plsc.store_scatter(orow_v, [sidx0 + g], acc0)
            plsc.store_scatter(orow_v, [sidx1 + g], acc1)

        pltpu.sync_copy(orow_v, out_hbm.at[row])


_sc_kernel = functools.partial(
    pl.kernel,
    out_type=jax.ShapeDtypeStruct((_B, _OUT_W), jnp.float32),
    mesh=plsc.VectorSubcoreMesh(core_axis_name="c", subcore_axis_name="s"),
    scratch_types=[
        pltpu.VMEM((_T,), jnp.int32),
        pltpu.VMEM((_T, _D), jnp.float32),
        pltpu.VMEM((_OUT_W,), jnp.float32),
        pltpu.SemaphoreType.DMA,
    ],
    compiler_params=pltpu.CompilerParams(
        use_tc_tiling_on_sc=False, needs_layout_passes=False),
)(_sc_body)


@jax.jit
def kernel(x, lut0, lut1, lut2):
    tab = jnp.concatenate([lut0, lut1, lut2], axis=1)
    return _sc_kernel(x, tab)


# speedup vs baseline: 45.1964x; 1.8119x over previous
import functools

import jax
import jax.numpy as jnp
from jax import lax
from jax.experimental import pallas as pl
from jax.experimental.pallas import tpu as pltpu
from jax.experimental.pallas import tpu_sc as plsc

_B = 4096
_H = 200
_F = 32
_CW = 3
_PW = 10
_OUT_LEN = _H - _CW + 1
_POOL_OUT = _OUT_LEN // _PW
_T = _POOL_OUT * _PW + _CW - 1
_D = _CW * _F
_NW = 32
_ROWS = _B // _NW
_OUT_W = _F * _POOL_OUT
_FLUSH = 16
_SB = _ROWS // (2 * _FLUSH)


def _sc_body(x_hbm, tab_hbm, out_hbm, xv, emb, oblk, sg0, sg1, so0, so1):
    wid = lax.axis_index("s") * 2 + lax.axis_index("c")
    base = wid * _ROWS
    lane = lax.iota(jnp.int32, 16)
    sidx0 = lane * _POOL_OUT
    sidx1 = (lane + 16) * _POOL_OUT
    sems_g = (sg0, sg1)
    sems_o = (so0, so1)

    pltpu.sync_copy(x_hbm.at[pl.ds(base, _ROWS)], xv)

    def fire_gather(r, slot):
        pltpu.async_copy(
            tab_hbm.at[xv.at[r, pl.ds(0, 96)]],
            emb.at[slot, pl.ds(0, 96)], sems_g[slot])
        pltpu.async_copy(
            tab_hbm.at[xv.at[r, pl.ds(96, 96)]],
            emb.at[slot, pl.ds(96, 96)], sems_g[slot])

    def wait_gather(slot):
        for c in range(2):
            pltpu.make_async_copy(
                tab_hbm.at[xv.at[0, pl.ds(c * 96, 96)]],
                emb.at[slot, pl.ds(c * 96, 96)], sems_g[slot]).wait()

    def flush_ref(sb, half):
        return out_hbm.at[pl.ds(base + sb * 32 + half * _FLUSH, _FLUSH)]

    fire_gather(0, 0)
    fire_gather(1, 1)

    @pl.loop(0, _SB)
    def _superblock(sb):
        for half in range(2):
            @pl.when(sb > 0)
            def _():
                pltpu.make_async_copy(
                    oblk.at[half], flush_ref(sb, half), sems_o[half]).wait()

            for k in range(_FLUSH):
                r = sb * 32 + half * _FLUSH + k
                slot = k % 2
                wait_gather(slot)

                @pl.loop(0, _POOL_OUT)
                def _grp(g):
                    j0 = g * _PW
                    acc0 = jnp.zeros((16,), jnp.float32)
                    acc1 = jnp.zeros((16,), jnp.float32)
                    for s in range(_PW):
                        j = j0 + s
                        e0 = emb[slot, j, pl.ds(0, 16)]
                        e1 = emb[slot, j, pl.ds(16, 16)]
                        f0 = emb[slot, j + 1, pl.ds(32, 16)]
                        f1 = emb[slot, j + 1, pl.ds(48, 16)]
                        g0 = emb[slot, j + 2, pl.ds(64, 16)]
                        g1 = emb[slot, j + 2, pl.ds(80, 16)]
                        acc0 = acc0 + jnp.maximum(e0 + f0 + g0, 0.0)
                        acc1 = acc1 + jnp.maximum(e1 + f1 + g1, 0.0)
                    krow = jnp.full((16,), k, jnp.int32)
                    plsc.store_scatter(
                        oblk.at[half], [krow, sidx0 + g], acc0)
                    plsc.store_scatter(
                        oblk.at[half], [krow, sidx1 + g], acc1)

                @pl.when(r + 2 < _ROWS)
                def _():
                    fire_gather(r + 2, slot)

            pltpu.async_copy(oblk.at[half], flush_ref(sb, half), sems_o[half])

    for half in range(2):
        pltpu.make_async_copy(
            oblk.at[half], flush_ref(_SB - 1, half), sems_o[half]).wait()


_sc_kernel = functools.partial(
    pl.kernel,
    out_type=jax.ShapeDtypeStruct((_B, _OUT_W), jnp.float32),
    mesh=plsc.VectorSubcoreMesh(core_axis_name="c", subcore_axis_name="s"),
    scratch_types=[
        pltpu.VMEM((_ROWS, _H), jnp.int32),
        pltpu.VMEM((2, _T, _D), jnp.float32),
        pltpu.VMEM((2, _FLUSH, _OUT_W), jnp.float32),
        pltpu.SemaphoreType.DMA,
        pltpu.SemaphoreType.DMA,
        pltpu.SemaphoreType.DMA,
        pltpu.SemaphoreType.DMA,
    ],
    compiler_params=pltpu.CompilerParams(
        use_tc_tiling_on_sc=False, needs_layout_passes=False),
)(_sc_body)


@jax.jit
def kernel(x, lut0, lut1, lut2):
    tab = jnp.concatenate([lut0, lut1, lut2], axis=1)
    return _sc_kernel(x, tab)


# speedup vs baseline: 56.0438x; 1.2400x over previous
import functools

import jax
import jax.numpy as jnp
from jax import lax
from jax.experimental import pallas as pl
from jax.experimental.pallas import tpu as pltpu
from jax.experimental.pallas import tpu_sc as plsc

_B = 4096
_H = 200
_F = 32
_CW = 3
_PW = 10
_OUT_LEN = _H - _CW + 1
_POOL_OUT = _OUT_LEN // _PW
_T = _POOL_OUT * _PW + _CW - 1
_D = _CW * _F
_NW = 32
_ROWS = _B // _NW
_OUT_W = _F * _POOL_OUT
_FLUSH = 16
_SB = _ROWS // (2 * _FLUSH)


def _sc_body(x_hbm, tab_hbm, out_hbm, xv, emb, oblk, sg0, sg1, so0, so1):
    wid = lax.axis_index("s") * 2 + lax.axis_index("c")
    base = wid * _ROWS
    lane = lax.iota(jnp.int32, 16)
    sidx_e = (2 * lane) * _POOL_OUT
    sidx_o = (2 * lane + 1) * _POOL_OUT
    sems_g = (sg0, sg1)
    sems_o = (so0, so1)

    pltpu.sync_copy(x_hbm.at[pl.ds(base, _ROWS)], xv)

    def fire_gather(r, slot):
        pltpu.async_copy(
            tab_hbm.at[xv.at[r, pl.ds(0, 96)]],
            emb.at[slot, pl.ds(0, 96)], sems_g[slot])
        pltpu.async_copy(
            tab_hbm.at[xv.at[r, pl.ds(96, 96)]],
            emb.at[slot, pl.ds(96, 96)], sems_g[slot])

    def wait_gather(slot):
        for c in range(2):
            pltpu.make_async_copy(
                tab_hbm.at[xv.at[0, pl.ds(c * 96, 96)]],
                emb.at[slot, pl.ds(c * 96, 96)], sems_g[slot]).wait()

    def flush_ref(sb, half):
        return out_hbm.at[pl.ds(base + sb * 32 + half * _FLUSH, _FLUSH)]

    fire_gather(0, 0)
    fire_gather(1, 1)

    @pl.loop(0, _SB)
    def _superblock(sb):
        for half in range(2):
            @pl.when(sb > 0)
            def _():
                pltpu.make_async_copy(
                    oblk.at[half], flush_ref(sb, half), sems_o[half]).wait()

            for k in range(_FLUSH):
                r = sb * 32 + half * _FLUSH + k
                slot = k % 2
                wait_gather(slot)

                @pl.loop(0, _POOL_OUT)
                def _grp(g):
                    j0 = g * _PW
                    acc = jnp.zeros((32,), jnp.bfloat16)
                    for s in range(_PW):
                        j = j0 + s
                        e = emb[slot, j, pl.ds(0, 32)]
                        f = emb[slot, j + 1, pl.ds(32, 32)]
                        h = emb[slot, j + 2, pl.ds(64, 32)]
                        acc = acc + jnp.maximum(e + f + h, 0.0)
                    krow = jnp.full((16,), k, jnp.int32)
                    a_e, a_o = plsc.unpack(
                        acc, format=plsc.PackFormat.INTERLEAVED)
                    plsc.store_scatter(
                        oblk.at[half], [krow, sidx_e + g], a_e)
                    plsc.store_scatter(
                        oblk.at[half], [krow, sidx_o + g], a_o)

                @pl.when(r + 2 < _ROWS)
                def _():
                    fire_gather(r + 2, slot)

            pltpu.async_copy(oblk.at[half], flush_ref(sb, half), sems_o[half])

    for half in range(2):
        pltpu.make_async_copy(
            oblk.at[half], flush_ref(_SB - 1, half), sems_o[half]).wait()


_sc_kernel = functools.partial(
    pl.kernel,
    out_type=jax.ShapeDtypeStruct((_B, _OUT_W), jnp.float32),
    mesh=plsc.VectorSubcoreMesh(core_axis_name="c", subcore_axis_name="s"),
    scratch_types=[
        pltpu.VMEM((_ROWS, _H), jnp.int32),
        pltpu.VMEM((2, _T, _D), jnp.bfloat16),
        pltpu.VMEM((2, _FLUSH, _OUT_W), jnp.float32),
        pltpu.SemaphoreType.DMA,
        pltpu.SemaphoreType.DMA,
        pltpu.SemaphoreType.DMA,
        pltpu.SemaphoreType.DMA,
    ],
    compiler_params=pltpu.CompilerParams(
        use_tc_tiling_on_sc=False, needs_layout_passes=False),
)(_sc_body)


@jax.jit
def kernel(x, lut0, lut1, lut2):
    tab = jnp.concatenate([lut0, lut1, lut2], axis=1).astype(jnp.bfloat16)
    return _sc_kernel(x, tab)


# speedup vs baseline: 63.6554x; 1.1358x over previous
import functools

import jax
import jax.numpy as jnp
from jax import lax
from jax.experimental import pallas as pl
from jax.experimental.pallas import tpu as pltpu
from jax.experimental.pallas import tpu_sc as plsc

_B = 4096
_H = 200
_F = 32
_CW = 3
_PW = 10
_OUT_LEN = _H - _CW + 1
_POOL_OUT = _OUT_LEN // _PW
_T = _POOL_OUT * _PW + _CW - 1
_D = _CW * _F
_NW = 32
_ROWS = _B // _NW
_OUT_W = _F * _POOL_OUT
_FLUSH = 16
_SB = _ROWS // (2 * _FLUSH)


def _sc_body(x_hbm, tab_hbm, out_hbm, xv, emb, oblk, sg0, sg1, sg2, sg3, so0, so1):
    wid = lax.axis_index("s") * 2 + lax.axis_index("c")
    base = wid * _ROWS
    lane = lax.iota(jnp.int32, 16)
    sidx_e = (2 * lane) * _POOL_OUT
    sidx_o = (2 * lane + 1) * _POOL_OUT
    sems_g = (sg0, sg1, sg2, sg3)
    sems_o = (so0, so1)

    pltpu.sync_copy(x_hbm.at[pl.ds(base, _ROWS)], xv)

    def fire_gather(r, slot):
        pltpu.async_copy(
            tab_hbm.at[xv.at[r, pl.ds(0, 96)]],
            emb.at[slot, pl.ds(0, 96)], sems_g[slot])
        pltpu.async_copy(
            tab_hbm.at[xv.at[r, pl.ds(96, 96)]],
            emb.at[slot, pl.ds(96, 96)], sems_g[slot])

    def wait_gather(slot):
        for c in range(2):
            pltpu.make_async_copy(
                tab_hbm.at[xv.at[0, pl.ds(c * 96, 96)]],
                emb.at[slot, pl.ds(c * 96, 96)], sems_g[slot]).wait()

    def flush_ref(sb, half):
        return out_hbm.at[pl.ds(base + sb * 32 + half * _FLUSH, _FLUSH)]

    for p in range(4):
        fire_gather(p, p)

    @pl.loop(0, _SB)
    def _superblock(sb):
        for half in range(2):
            @pl.when(sb > 0)
            def _():
                pltpu.make_async_copy(
                    oblk.at[half], flush_ref(sb, half), sems_o[half]).wait()

            for k in range(_FLUSH):
                r = sb * 32 + half * _FLUSH + k
                slot = k % 4
                wait_gather(slot)

                @pl.loop(0, _POOL_OUT)
                def _grp(g):
                    j0 = g * _PW
                    acc = jnp.zeros((32,), jnp.bfloat16)
                    for s in range(_PW):
                        j = j0 + s
                        e = emb[slot, j, pl.ds(0, 32)]
                        f = emb[slot, j + 1, pl.ds(32, 32)]
                        h = emb[slot, j + 2, pl.ds(64, 32)]
                        acc = acc + jnp.maximum(e + f + h, 0.0)
                    krow = jnp.full((16,), k, jnp.int32)
                    a_e, a_o = plsc.unpack(
                        acc, format=plsc.PackFormat.INTERLEAVED)
                    plsc.store_scatter(
                        oblk.at[half], [krow, sidx_e + g], a_e)
                    plsc.store_scatter(
                        oblk.at[half], [krow, sidx_o + g], a_o)

                @pl.when(r + 4 < _ROWS)
                def _():
                    fire_gather(r + 4, slot)

            pltpu.async_copy(oblk.at[half], flush_ref(sb, half), sems_o[half])

    for half in range(2):
        pltpu.make_async_copy(
            oblk.at[half], flush_ref(_SB - 1, half), sems_o[half]).wait()


_sc_kernel = functools.partial(
    pl.kernel,
    out_type=jax.ShapeDtypeStruct((_B, _OUT_W), jnp.float32),
    mesh=plsc.VectorSubcoreMesh(core_axis_name="c", subcore_axis_name="s"),
    scratch_types=[
        pltpu.VMEM((_ROWS, _H), jnp.int32),
        pltpu.VMEM((4, _T, _D), jnp.bfloat16),
        pltpu.VMEM((2, _FLUSH, _OUT_W), jnp.float32),
        pltpu.SemaphoreType.DMA,
        pltpu.SemaphoreType.DMA,
        pltpu.SemaphoreType.DMA,
        pltpu.SemaphoreType.DMA,
        pltpu.SemaphoreType.DMA,
        pltpu.SemaphoreType.DMA,
    ],
    compiler_params=pltpu.CompilerParams(
        use_tc_tiling_on_sc=False, needs_layout_passes=False),
)(_sc_body)


@jax.jit
def kernel(x, lut0, lut1, lut2):
    tab = jnp.concatenate([lut0, lut1, lut2], axis=1).astype(jnp.bfloat16)
    return _sc_kernel(x, tab)


# speedup vs baseline: 63.9310x; 1.0043x over previous
import functools

import jax
import jax.numpy as jnp
from jax import lax
from jax.experimental import pallas as pl
from jax.experimental.pallas import tpu as pltpu
from jax.experimental.pallas import tpu_sc as plsc

_B = 4096
_H = 200
_F = 32
_CW = 3
_PW = 10
_OUT_LEN = _H - _CW + 1
_POOL_OUT = _OUT_LEN // _PW
_T = _POOL_OUT * _PW + _CW - 1
_D = _CW * _F
_NW = 32
_ROWS = _B // _NW
_OUT_W = _F * _POOL_OUT
_FLUSH = 16
_SB = _ROWS // (2 * _FLUSH)


def _sc_body(x_hbm, tab_hbm, out_hbm, xv, emb, oblk, sg0, sg1, sg2, sg3, so0, so1):
    wid = lax.axis_index("s") * 2 + lax.axis_index("c")
    base = wid * _ROWS
    lane = lax.iota(jnp.int32, 16)
    sidx_e = (2 * lane) * _POOL_OUT
    sidx_o = (2 * lane + 1) * _POOL_OUT
    sems_g = (sg0, sg1, sg2, sg3)
    sems_o = (so0, so1)

    pltpu.sync_copy(x_hbm.at[pl.ds(base, _ROWS)], xv)

    def fire_gather(r, slot):
        pltpu.async_copy(
            tab_hbm.at[xv.at[r, pl.ds(0, 96)]],
            emb.at[slot, pl.ds(0, 96)], sems_g[slot])
        pltpu.async_copy(
            tab_hbm.at[xv.at[r, pl.ds(96, 96)]],
            emb.at[slot, pl.ds(96, 96)], sems_g[slot])

    def wait_gather(slot):
        pltpu.make_async_copy(
            tab_hbm.at[pl.ds(0, _T)], emb.at[slot], sems_g[slot]).wait()

    def flush_ref(sb, half):
        return out_hbm.at[pl.ds(base + sb * 32 + half * _FLUSH, _FLUSH)]

    for p in range(4):
        fire_gather(p, p)

    @pl.loop(0, _SB)
    def _superblock(sb):
        for half in range(2):
            @pl.when(sb > 0)
            def _():
                pltpu.make_async_copy(
                    oblk.at[half], flush_ref(sb, half), sems_o[half]).wait()

            for k in range(_FLUSH):
                r = sb * 32 + half * _FLUSH + k
                slot = k % 4
                wait_gather(slot)

                @pl.loop(0, _POOL_OUT)
                def _grp(g):
                    j0 = g * _PW
                    acc_a = jnp.zeros((32,), jnp.bfloat16)
                    acc_b = jnp.zeros((32,), jnp.bfloat16)
                    for s in range(0, _PW, 2):
                        j = j0 + s
                        e = emb[slot, j, pl.ds(0, 32)]
                        f = emb[slot, j + 1, pl.ds(32, 32)]
                        h = emb[slot, j + 2, pl.ds(64, 32)]
                        acc_a = acc_a + jnp.maximum(e + f + h, 0.0)
                        e = emb[slot, j + 1, pl.ds(0, 32)]
                        f = emb[slot, j + 2, pl.ds(32, 32)]
                        h = emb[slot, j + 3, pl.ds(64, 32)]
                        acc_b = acc_b + jnp.maximum(e + f + h, 0.0)
                    krow = jnp.full((16,), k, jnp.int32)
                    a_e, a_o = plsc.unpack(
                        acc_a + acc_b, format=plsc.PackFormat.INTERLEAVED)
                    plsc.store_scatter(
                        oblk.at[half], [krow, sidx_e + g], a_e)
                    plsc.store_scatter(
                        oblk.at[half], [krow, sidx_o + g], a_o)

                @pl.when(r + 4 < _ROWS)
                def _():
                    fire_gather(r + 4, slot)

            pltpu.async_copy(oblk.at[half], flush_ref(sb, half), sems_o[half])

    for half in range(2):
        pltpu.make_async_copy(
            oblk.at[half], flush_ref(_SB - 1, half), sems_o[half]).wait()


_sc_kernel = functools.partial(
    pl.kernel,
    out_type=jax.ShapeDtypeStruct((_B, _OUT_W), jnp.float32),
    mesh=plsc.VectorSubcoreMesh(core_axis_name="c", subcore_axis_name="s"),
    scratch_types=[
        pltpu.VMEM((_ROWS, _H), jnp.int32),
        pltpu.VMEM((4, _T, _D), jnp.bfloat16),
        pltpu.VMEM((2, _FLUSH, _OUT_W), jnp.float32),
        pltpu.SemaphoreType.DMA,
        pltpu.SemaphoreType.DMA,
        pltpu.SemaphoreType.DMA,
        pltpu.SemaphoreType.DMA,
        pltpu.SemaphoreType.DMA,
        pltpu.SemaphoreType.DMA,
    ],
    compiler_params=pltpu.CompilerParams(
        use_tc_tiling_on_sc=False, needs_layout_passes=False),
)(_sc_body)


@jax.jit
def kernel(x, lut0, lut1, lut2):
    tab = jnp.concatenate([lut0, lut1, lut2], axis=1).astype(jnp.bfloat16)
    return _sc_kernel(x, tab)


# speedup vs baseline: 70.7179x; 1.1062x over previous
import functools

import jax
import jax.numpy as jnp
from jax import lax
from jax.experimental import pallas as pl
from jax.experimental.pallas import tpu as pltpu
from jax.experimental.pallas import tpu_sc as plsc

_B = 4096
_V = 8192
_H = 200
_F = 32
_CW = 3
_PW = 10
_OUT_LEN = _H - _CW + 1
_POOL_OUT = _OUT_LEN // _PW
_T = _POOL_OUT * _PW + _CW - 1
_D = _CW * _F
_NW = 32
_ROWS = _B // _NW
_OUT_W = _F * _POOL_OUT
_FLUSH = 16
_SB = _ROWS // (2 * _FLUSH)


def _sc_body(x_hbm, tab_hbm, out_hbm, xv, emb, oblk, spm,
             sg0, sg1, sg2, sg3, so0, so1):
    sid = lax.axis_index("s")
    wid = sid * 2 + lax.axis_index("c")
    base = wid * _ROWS

    vrows = _V // 16
    pltpu.sync_copy(
        tab_hbm.at[pl.ds(sid * vrows, vrows)],
        spm.at[pl.ds(sid * vrows, vrows)])
    plsc.subcore_barrier()
    lane = lax.iota(jnp.int32, 16)
    sidx_e = (2 * lane) * _POOL_OUT
    sidx_o = (2 * lane + 1) * _POOL_OUT
    sems_g = (sg0, sg1, sg2, sg3)
    sems_o = (so0, so1)

    pltpu.sync_copy(x_hbm.at[pl.ds(base, _ROWS)], xv)

    def fire_gather(r, slot):
        pltpu.async_copy(
            spm.at[xv.at[r, pl.ds(0, 96)]],
            emb.at[slot, pl.ds(0, 96)], sems_g[slot])
        pltpu.async_copy(
            spm.at[xv.at[r, pl.ds(96, 96)]],
            emb.at[slot, pl.ds(96, 96)], sems_g[slot])

    def wait_gather(slot):
        pltpu.make_async_copy(
            spm.at[pl.ds(0, _T)], emb.at[slot], sems_g[slot]).wait()

    def flush_ref(sb, half):
        return out_hbm.at[pl.ds(base + sb * 32 + half * _FLUSH, _FLUSH)]

    for p in range(4):
        fire_gather(p, p)

    @pl.loop(0, _SB)
    def _superblock(sb):
        for half in range(2):
            @pl.when(sb > 0)
            def _():
                pltpu.make_async_copy(
                    oblk.at[half], flush_ref(sb, half), sems_o[half]).wait()

            for k in range(_FLUSH):
                r = sb * 32 + half * _FLUSH + k
                slot = k % 4
                wait_gather(slot)

                @pl.loop(0, _POOL_OUT)
                def _grp(g):
                    j0 = g * _PW
                    acc_a = jnp.zeros((32,), jnp.bfloat16)
                    acc_b = jnp.zeros((32,), jnp.bfloat16)
                    for s in range(0, _PW, 2):
                        j = j0 + s
                        e = emb[slot, j, pl.ds(0, 32)]
                        f = emb[slot, j + 1, pl.ds(32, 32)]
                        h = emb[slot, j + 2, pl.ds(64, 32)]
                        acc_a = acc_a + jnp.maximum(e + f + h, 0.0)
                        e = emb[slot, j + 1, pl.ds(0, 32)]
                        f = emb[slot, j + 2, pl.ds(32, 32)]
                        h = emb[slot, j + 3, pl.ds(64, 32)]
                        acc_b = acc_b + jnp.maximum(e + f + h, 0.0)
                    krow = jnp.full((16,), k, jnp.int32)
                    a_e, a_o = plsc.unpack(
                        acc_a + acc_b, format=plsc.PackFormat.INTERLEAVED)
                    plsc.store_scatter(
                        oblk.at[half], [krow, sidx_e + g], a_e)
                    plsc.store_scatter(
                        oblk.at[half], [krow, sidx_o + g], a_o)

                @pl.when(r + 4 < _ROWS)
                def _():
                    fire_gather(r + 4, slot)

            pltpu.async_copy(oblk.at[half], flush_ref(sb, half), sems_o[half])

    for half in range(2):
        pltpu.make_async_copy(
            oblk.at[half], flush_ref(_SB - 1, half), sems_o[half]).wait()


_sc_kernel = functools.partial(
    pl.kernel,
    out_type=jax.ShapeDtypeStruct((_B, _OUT_W), jnp.float32),
    mesh=plsc.VectorSubcoreMesh(core_axis_name="c", subcore_axis_name="s"),
    scratch_types=[
        pltpu.VMEM((_ROWS, _H), jnp.int32),
        pltpu.VMEM((4, _T, _D), jnp.bfloat16),
        pltpu.VMEM((2, _FLUSH, _OUT_W), jnp.float32),
        pltpu.VMEM_SHARED((_V, _D), jnp.bfloat16),
        pltpu.SemaphoreType.DMA,
        pltpu.SemaphoreType.DMA,
        pltpu.SemaphoreType.DMA,
        pltpu.SemaphoreType.DMA,
        pltpu.SemaphoreType.DMA,
        pltpu.SemaphoreType.DMA,
    ],
    compiler_params=pltpu.CompilerParams(
        use_tc_tiling_on_sc=False, needs_layout_passes=False),
)(_sc_body)


@jax.jit
def kernel(x, lut0, lut1, lut2):
    tab = jnp.concatenate([lut0, lut1, lut2], axis=1).astype(jnp.bfloat16)
    return _sc_kernel(x, tab)


# speedup vs baseline: 73.3566x; 1.0373x over previous
import functools

import jax
import jax.numpy as jnp
from jax import lax
from jax.experimental import pallas as pl
from jax.experimental.pallas import tpu as pltpu
from jax.experimental.pallas import tpu_sc as plsc

_B = 4096
_V = 8192
_H = 200
_F = 32
_CW = 3
_PW = 10
_OUT_LEN = _H - _CW + 1
_POOL_OUT = _OUT_LEN // _PW
_T = _POOL_OUT * _PW + _CW - 1
_NW = 32
_ROWS = _B // _NW
_OUT_W = _F * _POOL_OUT
_FLUSH = 16
_SB = _ROWS // (2 * _FLUSH)
_CR = _T + _CW - 1


def _sc_body(x_hbm, tab_hbm, out_hbm, xv, conv, oblk, spm,
             sg0, sg1, sg2, sg3, so0, so1):
    sid = lax.axis_index("s")
    wid = sid * 2 + lax.axis_index("c")
    base = wid * _ROWS
    lane = lax.iota(jnp.int32, 16)
    sidx_e = (2 * lane) * _POOL_OUT
    sidx_o = (2 * lane + 1) * _POOL_OUT
    zero32 = jnp.zeros((32,), jnp.bfloat16)
    sems_g = (sg0, sg1, sg2, sg3)
    sems_o = (so0, so1)

    vrows = _V // 16
    for l in range(_CW):
        pltpu.sync_copy(
            tab_hbm.at[l, pl.ds(sid * vrows, vrows)],
            spm.at[l, pl.ds(sid * vrows, vrows)])

    @pl.loop(0, _CR)
    def _zero(i):
        for slot in range(4):
            conv[slot, i, :] = zero32

    plsc.subcore_barrier()

    pltpu.sync_copy(x_hbm.at[pl.ds(base, _ROWS)], xv)

    def fire_gather(r, slot):
        for l in range(_CW):
            for c in range(2):
                pltpu.async_copy(
                    spm.at[l].at[xv.at[r, pl.ds(96 * c, 96)]],
                    conv.at[slot, pl.ds(2 - l + 96 * c, 96)],
                    sems_g[slot], add=True)

    def wait_gather(slot):
        for _ in range(3):
            pltpu.make_async_copy(
                spm.at[0, pl.ds(0, 2 * 96)],
                conv.at[slot, pl.ds(0, 2 * 96)], sems_g[slot]).wait()

    def flush_ref(sb, half):
        return out_hbm.at[pl.ds(base + sb * 32 + half * _FLUSH, _FLUSH)]

    for p in range(4):
        fire_gather(p, p)

    @pl.loop(0, _SB)
    def _superblock(sb):
        for half in range(2):
            @pl.when(sb > 0)
            def _():
                pltpu.make_async_copy(
                    oblk.at[half], flush_ref(sb, half), sems_o[half]).wait()

            for k in range(_FLUSH):
                r = sb * 32 + half * _FLUSH + k
                slot = k % 4
                wait_gather(slot)

                @pl.loop(0, _POOL_OUT)
                def _grp(g):
                    j0 = g * _PW + 2
                    acc_a = jnp.zeros((32,), jnp.bfloat16)
                    acc_b = jnp.zeros((32,), jnp.bfloat16)
                    for s in range(0, _PW, 2):
                        ca = conv[slot, j0 + s, :]
                        acc_a = acc_a + jnp.maximum(ca, 0)
                        conv[slot, j0 + s, :] = zero32
                        cb = conv[slot, j0 + s + 1, :]
                        acc_b = acc_b + jnp.maximum(cb, 0)
                        conv[slot, j0 + s + 1, :] = zero32
                    krow = jnp.full((16,), k, jnp.int32)
                    a_e, a_o = plsc.unpack(
                        acc_a + acc_b, format=plsc.PackFormat.INTERLEAVED)
                    plsc.store_scatter(
                        oblk.at[half], [krow, sidx_e + g], a_e)
                    plsc.store_scatter(
                        oblk.at[half], [krow, sidx_o + g], a_o)

                @pl.when(r + 4 < _ROWS)
                def _():
                    fire_gather(r + 4, slot)

            pltpu.async_copy(oblk.at[half], flush_ref(sb, half), sems_o[half])

    for half in range(2):
        pltpu.make_async_copy(
            oblk.at[half], flush_ref(_SB - 1, half), sems_o[half]).wait()


_sc_kernel = functools.partial(
    pl.kernel,
    out_type=jax.ShapeDtypeStruct((_B, _OUT_W), jnp.float32),
    mesh=plsc.VectorSubcoreMesh(core_axis_name="c", subcore_axis_name="s"),
    scratch_types=[
        pltpu.VMEM((_ROWS, _H), jnp.int32),
        pltpu.VMEM((4, _CR, _F), jnp.bfloat16),
        pltpu.VMEM((2, _FLUSH, _OUT_W), jnp.float32),
        pltpu.VMEM_SHARED((_CW, _V, _F), jnp.bfloat16),
        pltpu.SemaphoreType.DMA,
        pltpu.SemaphoreType.DMA,
        pltpu.SemaphoreType.DMA,
        pltpu.SemaphoreType.DMA,
        pltpu.SemaphoreType.DMA,
        pltpu.SemaphoreType.DMA,
    ],
    compiler_params=pltpu.CompilerParams(
        use_tc_tiling_on_sc=False, needs_layout_passes=False),
)(_sc_body)


@jax.jit
def kernel(x, lut0, lut1, lut2):
    tab = jnp.stack([lut0, lut1, lut2]).astype(jnp.bfloat16)
    return _sc_kernel(x, tab)
